# baseline (speedup 1.0000x reference)
import jax
import jax.numpy as jnp
from jax import lax
from jax.experimental import pallas as pl
from jax.experimental.pallas import tpu as pltpu
from jax.experimental.pallas import tpu_sc as plsc

_NC = 2
_NS = 16
_NW = _NC * _NS
_NBUF = 8
_W = (96, 104)
_OFF = (0, 96)


def _make_body(SEQ_W, CH):
    def body(x_hbm, tok_hbm, pos_hbm, out_hbm, idx_a, idx_b, rv, si, sg, sw):
        wid = lax.axis_index("s") * _NC + lax.axis_index("c")
        seq0 = wid * SEQ_W
        pltpu.sync_copy(x_hbm.at[pl.ds(seq0, SEQ_W), pl.ds(_OFF[0], _W[0])],
                        idx_a)
        pltpu.sync_copy(x_hbm.at[pl.ds(seq0, SEQ_W), pl.ds(_OFF[1], _W[1])],
                        idx_b)
        idx = (idx_a, idx_b)

        def init(c, h, b):
            pltpu.async_copy(pos_hbm.at[pl.ds(_OFF[h], _W[h])],
                             rv.at[b, pl.ds(0, _W[h])], si.at[b])

        def init_wait(c, h, b):
            pltpu.make_async_copy(pos_hbm.at[pl.ds(_OFF[h], _W[h])],
                                  rv.at[b, pl.ds(0, _W[h])], si.at[b]).wait()

        def gather(c, h, b):
            pltpu.async_copy(tok_hbm.at[idx[h].at[c // 2]],
                             rv.at[b, pl.ds(0, _W[h])], sg.at[b], add=True)

        def gather_wait(c, h, b):
            pltpu.make_async_copy(tok_hbm.at[idx[h].at[c // 2]],
                                  rv.at[b, pl.ds(0, _W[h])], sg.at[b]).wait()

        def write(c, h, b):
            pltpu.async_copy(
                rv.at[b, pl.ds(0, _W[h])],
                out_hbm.at[seq0 + c // 2, pl.ds(_OFF[h], _W[h])], sw.at[b])

        def write_wait(c, h, b):
            pltpu.make_async_copy(
                rv.at[b, pl.ds(0, _W[h])],
                out_hbm.at[seq0 + c // 2, pl.ds(_OFF[h], _W[h])],
                sw.at[b]).wait()

        for c0 in range(4):
            init(c0, c0 % 2, c0)
        init_wait(0, 0, 0)
        gather(0, 0, 0)
        init_wait(1, 1, 1)
        gather(1, 1, 1)

        @pl.loop(0, CH, step=_NBUF)
        def _(t):
            for k in range(_NBUF):
                c = t + k
                h = k % 2
                b2 = (k + 2) % _NBUF
                b4 = (k + 4) % _NBUF

                @pl.when(c + 4 < CH)
                def _():
                    @pl.when(c >= 4)
                    def _():
                        write_wait(c - 4, h, b4)

                    init(c + 4, h, b4)

                @pl.when(c + 2 < CH)
                def _():
                    init_wait(c + 2, h, b2)
                    gather(c + 2, h, b2)

                gather_wait(c, h, k)
                write(c, h, k)

        for k in range(_NBUF):
            write_wait(CH - _NBUF + k, k % 2, k)

    return body


def kernel(x, token_table, pos_table):
    B, L = x.shape
    V, D = token_table.shape
    SEQ_W = B // _NW
    CH = 2 * SEQ_W

    mesh = plsc.VectorSubcoreMesh(core_axis_name="c", subcore_axis_name="s")
    out = pl.kernel(
        _make_body(SEQ_W, CH),
        out_type=jax.ShapeDtypeStruct((B, L, D), jnp.float32),
        mesh=mesh,
        compiler_params=pltpu.CompilerParams(use_tc_tiling_on_sc=False),
        scratch_types=[
            pltpu.VMEM((SEQ_W, _W[0]), jnp.int32),
            pltpu.VMEM((SEQ_W, _W[1]), jnp.int32),
            pltpu.VMEM((_NBUF, _W[1], D), jnp.float32),
            pltpu.SemaphoreType.DMA((_NBUF,)),
            pltpu.SemaphoreType.DMA((_NBUF,)),
            pltpu.SemaphoreType.DMA((_NBUF,)),
        ],
    )(x, token_table, pos_table)
    return out


# speedup vs baseline: 1.2490x; 1.2490x over previous
import math

import jax
import jax.numpy as jnp
from jax import lax
from jax.experimental import pallas as pl
from jax.experimental.pallas import tpu as pltpu
from jax.experimental.pallas import tpu_sc as plsc

_NC = 2
_NS = 16
_NW = _NC * _NS
_CHUNK = 128
_NBUF = 8
_K = 5


def _make_body(CH, ROWS_W, PERIOD, CHUNK_OFF):
    def body(x_hbm, tok_hbm, pose_hbm, out_hbm, idx_v, rv, si, sg, sw):
        wid = lax.axis_index("s") * _NC + lax.axis_index("c")
        pltpu.sync_copy(x_hbm.at[wid], idx_v)
        base = wid * ROWS_W
        ph = CHUNK_OFF + wid * CH

        def init(c, b):
            pltpu.async_copy(pose_hbm.at[lax.rem(ph + c, PERIOD)], rv.at[b],
                             si.at[b])

        def init_wait(c, b):
            pltpu.make_async_copy(pose_hbm.at[lax.rem(ph + c, PERIOD)],
                                  rv.at[b], si.at[b]).wait()

        def gather(c, b):
            pltpu.async_copy(tok_hbm.at[idx_v.at[c]], rv.at[b], sg.at[b],
                             add=True)

        def gather_wait(c, b):
            pltpu.make_async_copy(
                tok_hbm.at[idx_v.at[c]], rv.at[b], sg.at[b]).wait()

        def write(c, b):
            pltpu.async_copy(
                rv.at[b], out_hbm.at[pl.ds(base + c * _CHUNK, _CHUNK)],
                sw.at[b])

        def write_wait(c, b):
            pltpu.make_async_copy(
                rv.at[b], out_hbm.at[pl.ds(base + c * _CHUNK, _CHUNK)],
                sw.at[b]).wait()

        for c0 in range(4):
            init(c0, c0)
        init_wait(0, 0)
        gather(0, 0)
        init_wait(1, 1)
        gather(1, 1)

        @pl.loop(0, CH, step=_NBUF)
        def _(t):
            for k in range(_NBUF):
                c = t + k
                b2 = (k + 2) % _NBUF
                b4 = (k + 4) % _NBUF

                @pl.when(c + 4 < CH)
                def _():
                    @pl.when(c >= 4)
                    def _():
                        write_wait(c - 4, b4)

                    init(c + 4, b4)

                @pl.when(c + 2 < CH)
                def _():
                    init_wait(c + 2, b2)
                    gather(c + 2, b2)

                gather_wait(c, k)
                write(c, k)

        for k in range(_NBUF):
            write_wait(CH - _NBUF + k, k)

    return body


def kernel(x, token_table, pos_table):
    B, L = x.shape
    V, D = token_table.shape
    N = B * L
    NP = N // _K
    ROWS_W = NP // _NW
    CH = ROWS_W // _CHUNK
    PERIOD = math.lcm(_CHUNK, L) // _CHUNK

    x_r = x.reshape(_K, _NW, CH, _CHUNK)
    reps = PERIOD * _CHUNK // L
    pos_exp = jnp.tile(pos_table, (reps, 1)).reshape(PERIOD, _CHUNK, D)

    mesh = plsc.VectorSubcoreMesh(core_axis_name="c", subcore_axis_name="s")
    parts = []
    for p in range(_K):
        part = pl.kernel(
            _make_body(CH, ROWS_W, PERIOD, p * (NP // _CHUNK)),
            out_type=jax.ShapeDtypeStruct((NP, D), jnp.float32),
            mesh=mesh,
            compiler_params=pltpu.CompilerParams(use_tc_tiling_on_sc=False),
            scratch_types=[
                pltpu.VMEM((CH, _CHUNK), jnp.int32),
                pltpu.VMEM((_NBUF, _CHUNK, D), jnp.float32),
                pltpu.SemaphoreType.DMA((_NBUF,)),
                pltpu.SemaphoreType.DMA((_NBUF,)),
                pltpu.SemaphoreType.DMA((_NBUF,)),
            ],
        )(x_r[p], token_table, pos_exp)
        parts.append(part)

    out = jnp.concatenate(parts, axis=0)
    return out.reshape(B, L, D)


# speedup vs baseline: 1.5025x; 1.2029x over previous
import math

import jax
import jax.numpy as jnp
from jax import lax
from jax.experimental import pallas as pl
from jax.experimental.pallas import tpu as pltpu
from jax.experimental.pallas import tpu_sc as plsc

_NC = 2
_NS = 16
_NW = _NC * _NS
_CHUNK = 128
_NBUF = 10


def _make_body(CH, ROWS_W, PERIOD):
    def body(x_hbm, tok_hbm, pose_hbm, out_hbm, idx_v, rv, si, sg, sw):
        wid = lax.axis_index("s") * _NC + lax.axis_index("c")
        pltpu.sync_copy(x_hbm.at[wid], idx_v)
        base = wid * ROWS_W

        def init(c, b):
            pltpu.async_copy(pose_hbm.at[lax.rem(c, PERIOD)], rv.at[b],
                             si.at[b])

        def init_wait(c, b):
            pltpu.make_async_copy(pose_hbm.at[lax.rem(c, PERIOD)], rv.at[b],
                                  si.at[b]).wait()

        def gather(c, b):
            pltpu.async_copy(tok_hbm.at[idx_v.at[c]], rv.at[b], sg.at[b],
                             add=True)

        def gather_wait(c, b):
            pltpu.make_async_copy(
                tok_hbm.at[idx_v.at[c]], rv.at[b], sg.at[b]).wait()

        def write(c, b):
            pltpu.async_copy(
                rv.at[b], out_hbm.at[pl.ds(base + c * _CHUNK, _CHUNK)],
                sw.at[b])

        def write_wait(c, b):
            pltpu.make_async_copy(
                rv.at[b], out_hbm.at[pl.ds(base + c * _CHUNK, _CHUNK)],
                sw.at[b]).wait()

        for c0 in range(4):
            init(c0, c0)
        init_wait(0, 0)
        gather(0, 0)
        init_wait(1, 1)
        gather(1, 1)

        @pl.loop(0, CH, step=_NBUF)
        def _(t):
            for k in range(_NBUF):
                c = t + k
                b2 = (k + 2) % _NBUF
                b4 = (k + 4) % _NBUF

                @pl.when(c + 4 < CH)
                def _():
                    @pl.when(c >= _NBUF - 4)
                    def _():
                        write_wait(c + 4 - _NBUF, b4)

                    init(c + 4, b4)

                @pl.when(c + 2 < CH)
                def _():
                    init_wait(c + 2, b2)
                    gather(c + 2, b2)

                gather_wait(c, k)
                write(c, k)

        for k in range(_NBUF):
            write_wait(CH - _NBUF + k, k)

    return body


def kernel(x, token_table, pos_table):
    B, L = x.shape
    V, D = token_table.shape
    N = B * L
    ROWS_W = N // _NW
    CH = ROWS_W // _CHUNK
    PERIOD = math.lcm(_CHUNK, L) // _CHUNK

    x_r = x.reshape(_NW, CH, _CHUNK)
    reps = PERIOD * _CHUNK // L
    pos_exp = jnp.tile(pos_table, (reps, 1)).reshape(PERIOD, _CHUNK, D)

    mesh = plsc.VectorSubcoreMesh(core_axis_name="c", subcore_axis_name="s")
    out = pl.kernel(
        _make_body(CH, ROWS_W, PERIOD),
        out_type=jax.ShapeDtypeStruct((N, D), jnp.float32),
        mesh=mesh,
        compiler_params=pltpu.CompilerParams(use_tc_tiling_on_sc=False),
        scratch_types=[
            pltpu.VMEM((CH, _CHUNK), jnp.int32),
            pltpu.VMEM((_NBUF, _CHUNK, D), jnp.float32),
            pltpu.SemaphoreType.DMA((_NBUF,)),
            pltpu.SemaphoreType.DMA((_NBUF,)),
            pltpu.SemaphoreType.DMA((_NBUF,)),
        ],
    )(x_r, token_table, pos_exp)
    return out.reshape(B, L, D)


# speedup vs baseline: 1.6750x; 1.1148x over previous
import math

import jax
import jax.numpy as jnp
from jax import lax
from jax.experimental import pallas as pl
from jax.experimental.pallas import tpu as pltpu
from jax.experimental.pallas import tpu_sc as plsc

_NC = 2
_NS = 16
_NW = _NC * _NS
_CHUNK = 64
_NBUF = 8
_PD = 128


def _make_body(CH, ROWS_W, PERIOD):
    def body(x_hbm, tok_hbm, pose_hbm, out_hbm, idx_v, rv, si, sg, sw):
        wid = lax.axis_index("s") * _NC + lax.axis_index("c")
        pltpu.sync_copy(x_hbm.at[wid], idx_v)
        base = wid * ROWS_W

        def init(c, b):
            pltpu.async_copy(pose_hbm.at[lax.rem(c, PERIOD)], rv.at[b],
                             si.at[b])

        def init_wait(c, b):
            pltpu.make_async_copy(pose_hbm.at[lax.rem(c, PERIOD)], rv.at[b],
                                  si.at[b]).wait()

        def gather(c, b):
            pltpu.async_copy(tok_hbm.at[idx_v.at[c]], rv.at[b], sg.at[b],
                             add=True)

        def gather_wait(c, b):
            pltpu.make_async_copy(
                tok_hbm.at[idx_v.at[c]], rv.at[b], sg.at[b]).wait()

        def write(c, b):
            pltpu.async_copy(
                rv.at[b], out_hbm.at[pl.ds(base + c * _CHUNK, _CHUNK)],
                sw.at[b])

        def write_wait(c, b):
            pltpu.make_async_copy(
                rv.at[b], out_hbm.at[pl.ds(base + c * _CHUNK, _CHUNK)],
                sw.at[b]).wait()

        for c0 in range(4):
            init(c0, c0)
        init_wait(0, 0)
        gather(0, 0)
        init_wait(1, 1)
        gather(1, 1)

        @pl.loop(0, CH, step=_NBUF)
        def _(t):
            for k in range(_NBUF):
                c = t + k
                b2 = (k + 2) % _NBUF
                b4 = (k + 4) % _NBUF

                @pl.when(c + 4 < CH)
                def _():
                    @pl.when(c >= _NBUF - 4)
                    def _():
                        write_wait(c + 4 - _NBUF, b4)

                    init(c + 4, b4)

                @pl.when(c + 2 < CH)
                def _():
                    init_wait(c + 2, b2)
                    gather(c + 2, b2)

                gather_wait(c, k)
                write(c, k)

        for k in range(_NBUF):
            write_wait(CH - _NBUF + k, k)

    return body


def kernel(x, token_table, pos_table):
    B, L = x.shape
    V, D = token_table.shape
    N = B * L
    ROWS_W = N // _NW
    CH = ROWS_W // _CHUNK
    PERIOD = math.lcm(_CHUNK, L) // _CHUNK

    x_r = x.reshape(_NW, CH, _CHUNK)
    tok_pad = jnp.pad(token_table, ((0, 0), (0, _PD - D)))
    reps = PERIOD * _CHUNK // L
    pos_exp = jnp.pad(
        jnp.tile(pos_table, (reps, 1)).reshape(PERIOD, _CHUNK, D),
        ((0, 0), (0, 0), (0, _PD - D)))

    mesh = plsc.VectorSubcoreMesh(core_axis_name="c", subcore_axis_name="s")
    out = pl.kernel(
        _make_body(CH, ROWS_W, PERIOD),
        out_type=jax.ShapeDtypeStruct((N, _PD), jnp.float32),
        mesh=mesh,
        scratch_types=[
            pltpu.VMEM((CH, _CHUNK), jnp.int32),
            pltpu.VMEM((_NBUF, _CHUNK, _PD), jnp.float32),
            pltpu.SemaphoreType.DMA((_NBUF,)),
            pltpu.SemaphoreType.DMA((_NBUF,)),
            pltpu.SemaphoreType.DMA((_NBUF,)),
        ],
    )(x_r, tok_pad, pos_exp)
    return out[:, :D].reshape(B, L, D)


# speedup vs baseline: 2.2946x; 1.3699x over previous
import math

import jax
import jax.numpy as jnp
from jax import lax
from jax.experimental import pallas as pl
from jax.experimental.pallas import tpu as pltpu
from jax.experimental.pallas import tpu_sc as plsc

_NC = 2
_NS = 16
_NW = _NC * _NS
_CHUNK = 64
_NBUF = 8
_PD = 128


def _make_body(CH, ROWS_W, PERIOD):
    def body(x_hbm, tok_hbm, pose_hbm, out_hbm, idx_v, rv, pos_sh,
             si, sg, sw):
        sid = lax.axis_index("s")
        wid = sid * _NC + lax.axis_index("c")
        @pl.when(sid == 0)
        def _():
            pltpu.sync_copy(pose_hbm, pos_sh)

        pltpu.sync_copy(x_hbm.at[wid], idx_v)
        plsc.subcore_barrier()
        base = wid * ROWS_W

        def init(c, b):
            pltpu.async_copy(pos_sh.at[lax.rem(c, PERIOD)], rv.at[b],
                             si.at[b])

        def init_wait(c, b):
            pltpu.make_async_copy(pos_sh.at[lax.rem(c, PERIOD)], rv.at[b],
                                  si.at[b]).wait()

        def gather(c, b):
            pltpu.async_copy(tok_hbm.at[idx_v.at[c]], rv.at[b], sg.at[b],
                             add=True)

        def gather_wait(c, b):
            pltpu.make_async_copy(
                tok_hbm.at[idx_v.at[c]], rv.at[b], sg.at[b]).wait()

        def write(c, b):
            pltpu.async_copy(
                rv.at[b], out_hbm.at[pl.ds(base + c * _CHUNK, _CHUNK)],
                sw.at[b])

        def write_wait(c, b):
            pltpu.make_async_copy(
                rv.at[b], out_hbm.at[pl.ds(base + c * _CHUNK, _CHUNK)],
                sw.at[b]).wait()

        for c0 in range(4):
            init(c0, c0)
        init_wait(0, 0)
        gather(0, 0)
        init_wait(1, 1)
        gather(1, 1)

        @pl.loop(0, CH, step=_NBUF)
        def _(t):
            for k in range(_NBUF):
                c = t + k
                b2 = (k + 2) % _NBUF
                b4 = (k + 4) % _NBUF

                @pl.when(c + 4 < CH)
                def _():
                    @pl.when(c >= _NBUF - 4)
                    def _():
                        write_wait(c + 4 - _NBUF, b4)

                    init(c + 4, b4)

                @pl.when(c + 2 < CH)
                def _():
                    init_wait(c + 2, b2)
                    gather(c + 2, b2)

                gather_wait(c, k)
                write(c, k)

        for k in range(_NBUF):
            write_wait(CH - _NBUF + k, k)

    return body


def kernel(x, token_table, pos_table):
    B, L = x.shape
    V, D = token_table.shape
    N = B * L
    ROWS_W = N // _NW
    CH = ROWS_W // _CHUNK
    PERIOD = math.lcm(_CHUNK, L) // _CHUNK

    x_r = x.reshape(_NW, CH, _CHUNK)
    tok_pad = jnp.pad(token_table, ((0, 0), (0, _PD - D)))
    reps = PERIOD * _CHUNK // L
    pos_exp = jnp.pad(
        jnp.tile(pos_table, (reps, 1)).reshape(PERIOD, _CHUNK, D),
        ((0, 0), (0, 0), (0, _PD - D)))

    mesh = plsc.VectorSubcoreMesh(core_axis_name="c", subcore_axis_name="s")
    out = pl.kernel(
        _make_body(CH, ROWS_W, PERIOD),
        out_type=jax.ShapeDtypeStruct((N, _PD), jnp.float32),
        mesh=mesh,
        scratch_types=[
            pltpu.VMEM((CH, _CHUNK), jnp.int32),
            pltpu.VMEM((_NBUF, _CHUNK, _PD), jnp.float32),
            pltpu.VMEM_SHARED((25, _CHUNK, _PD), jnp.float32),
            pltpu.SemaphoreType.DMA((_NBUF,)),
            pltpu.SemaphoreType.DMA((_NBUF,)),
            pltpu.SemaphoreType.DMA((_NBUF,)),
        ],
    )(x_r, tok_pad, pos_exp)
    return out[:, :D].reshape(B, L, D)
